# baseline (speedup 1.0000x reference)
import jax
import jax.numpy as jnp
from jax import lax
from jax.experimental import pallas as pl
from jax.experimental.pallas import tpu as pltpu
from jax.experimental.pallas import tpu_sc as plsc

SEQ = 8192
DM = 1024
NCLS = 42
NCLS_PAD = 48
NSUB = 16
KCH = 2 * DM // NSUB
MCH = SEQ // NSUB
SUBROW = 128
ROWS_PER = DM // SUBROW
NEG_BIG = -1e30


def _iota16():
    return lax.broadcasted_iota(jnp.int32, (16,), 0)


def _lane_allsum(v, scratch):
    for s in (1, 2, 4, 8):
        scratch[...] = v
        v = v + plsc.load_gather(scratch, [jnp.bitwise_xor(_iota16(), s)])
    return v


def _lane_allmax(v, scratch):
    for s in (1, 2, 4, 8):
        scratch[...] = v
        v = jnp.maximum(
            v, plsc.load_gather(scratch, [jnp.bitwise_xor(_iota16(), s)]))
    return v


ROWW = 80


def _body(bert_hbm, e1_hbm, e2_hbm, w_hbm, b_hbm, out_hbm,
          w_v, x_v, m_v, pv_v, all_v, part_v, b_v, out_v, red_v,
          sh, sem_w, sem_g):
    cid = lax.axis_index("c")
    sid = lax.axis_index("s")
    iota = _iota16()
    zeros16 = jnp.zeros((16,), jnp.int32)

    pltpu.sync_copy(e1_hbm.at[pl.ds(sid * MCH, MCH)], m_v.at[pl.ds(0, MCH)])
    pltpu.sync_copy(e2_hbm.at[pl.ds(sid * MCH, MCH)],
                    m_v.at[pl.ds(MCH, MCH)])
    wcopy = pltpu.async_copy(
        w_hbm.at[pl.ds(sid * (KCH * NCLS_PAD), KCH * NCLS_PAD)],
        w_v, sem_w)
    base = sid * MCH
    acc1 = zeros16
    acc2 = zeros16
    for j in range(MCH // 16):
        bv1 = m_v[pl.ds(16 * j, 16)] != 0
        bv2 = m_v[pl.ds(MCH + 16 * j, 16)] != 0
        pc1 = plsc.all_reduce_population_count(bv1)
        pc2 = plsc.all_reduce_population_count(bv2)
        f1 = plsc.all_reduce_ffs(bv1)
        f2 = plsc.all_reduce_ffs(bv2)
        acc1 = acc1 + jnp.where(pc1 > 0, f1 + (base + 16 * j), 0)
        acc2 = acc2 + jnp.where(pc2 > 0, f2 + (base + 16 * j), 0)
    pv_v[pl.ds(0, 16)] = acc1
    pv_v[pl.ds(16, 16)] = acc2
    pltpu.sync_copy(pv_v, sh.at[pl.ds(sid * ROWW, 32)])
    plsc.subcore_barrier()

    pltpu.sync_copy(sh, all_v)
    tot1 = zeros16
    tot2 = zeros16
    for j in range(NSUB):
        tot1 = tot1 + all_v[pl.ds(j * ROWW, 16)]
        tot2 = tot2 + all_v[pl.ds(j * ROWW + 16, 16)]

    half = NSUB // 2
    ridx = jnp.where(sid < half, tot1 * ROWS_PER + sid,
                     tot2 * ROWS_PER + (sid - half))
    pltpu.async_copy(bert_hbm.at[ridx], x_v, sem_g).wait()

    wcopy.wait()
    l0 = jnp.zeros((16,), jnp.float32)
    l1 = jnp.zeros((16,), jnp.float32)
    l2 = jnp.zeros((16,), jnp.float32)
    ones16 = jnp.full((16,), 1, jnp.int32)
    for k in range(KCH):
        xs = plsc.load_gather(
            x_v, [ones16, jnp.full((16,), k, jnp.int32)])
        woff = k * NCLS_PAD
        l0 = l0 + xs * w_v[pl.ds(woff, 16)]
        l1 = l1 + xs * w_v[pl.ds(woff + 16, 16)]
        l2 = l2 + xs * w_v[pl.ds(woff + 32, 16)]
    part_v[pl.ds(0, 16)] = plsc.bitcast(l0, jnp.int32)
    part_v[pl.ds(16, 16)] = plsc.bitcast(l1, jnp.int32)
    part_v[pl.ds(32, 16)] = plsc.bitcast(l2, jnp.int32)
    pltpu.sync_copy(part_v, sh.at[pl.ds(sid * ROWW + 32, 48)])
    plsc.subcore_barrier()

    @pl.when(jnp.logical_and(cid == 0, sid == 0))
    def _():
        pltpu.sync_copy(b_hbm, b_v)
        pltpu.sync_copy(sh, all_v)
        g0 = jnp.zeros((16,), jnp.float32)
        g1 = jnp.zeros((16,), jnp.float32)
        g2 = jnp.zeros((16,), jnp.float32)
        for j in range(NSUB):
            g0 = g0 + plsc.bitcast(all_v[pl.ds(j * ROWW + 32, 16)],
                                   jnp.float32)
            g1 = g1 + plsc.bitcast(all_v[pl.ds(j * ROWW + 48, 16)],
                                   jnp.float32)
            g2 = g2 + plsc.bitcast(all_v[pl.ds(j * ROWW + 64, 16)],
                                   jnp.float32)
        g0 = g0 + b_v[pl.ds(0, 16)]
        g1 = g1 + b_v[pl.ds(16, 16)]
        g2 = jnp.where(iota < NCLS - 32, g2 + b_v[pl.ds(32, 16)], NEG_BIG)
        m = _lane_allmax(jnp.maximum(jnp.maximum(g0, g1), g2), red_v)
        e0 = jnp.exp(g0 - m)
        e1 = jnp.exp(g1 - m)
        e2 = jnp.exp(g2 - m)
        s = _lane_allsum(e0 + e1 + e2, red_v)
        out_v[pl.ds(0, 16)] = e0 / s
        out_v[pl.ds(16, 16)] = e1 / s
        out_v[pl.ds(32, 16)] = e2 / s
        pltpu.sync_copy(out_v, out_hbm)


@jax.jit
def _entity_start_head_sc(bert2, e1i, e2i, w_pad, b_pad):
    mesh = plsc.VectorSubcoreMesh(core_axis_name="c", subcore_axis_name="s",
                                  num_cores=2, num_subcores=NSUB)
    return pl.kernel(
        _body,
        out_type=jax.ShapeDtypeStruct((NCLS_PAD,), jnp.float32),
        mesh=mesh,
        compiler_params=pltpu.CompilerParams(needs_layout_passes=False),
        scratch_types=[
            pltpu.VMEM((KCH * NCLS_PAD,), jnp.float32),
            pltpu.VMEM((16, SUBROW), jnp.float32),
            pltpu.VMEM((2 * MCH,), jnp.int32),
            pltpu.VMEM((32,), jnp.int32),
            pltpu.VMEM((NSUB * ROWW,), jnp.int32),
            pltpu.VMEM((NCLS_PAD,), jnp.int32),
            pltpu.VMEM((NCLS_PAD,), jnp.float32),
            pltpu.VMEM((NCLS_PAD,), jnp.float32),
            pltpu.VMEM((16,), jnp.float32),
            pltpu.VMEM_SHARED((NSUB * ROWW,), jnp.int32),
            pltpu.SemaphoreType.DMA,
            pltpu.SemaphoreType.DMA,
        ],
    )(bert2, e1i, e2i, w_pad, b_pad)


def kernel(bert_output, e1_mask, e2_mask, W, b):
    bert2 = bert_output.reshape(SEQ * ROWS_PER, SUBROW)
    e1i = e1_mask.reshape(SEQ).astype(jnp.int32)
    e2i = e2_mask.reshape(SEQ).astype(jnp.int32)
    w_pad = jnp.pad(W, ((0, 0), (0, NCLS_PAD - NCLS))).reshape(-1)
    b_pad = jnp.pad(b, (0, NCLS_PAD - NCLS))
    out = _entity_start_head_sc(bert2, e1i, e2i, w_pad, b_pad)
    return out[:NCLS].reshape(1, NCLS)


# speedup vs baseline: 1.0259x; 1.0259x over previous
import jax
import jax.numpy as jnp
from jax import lax
from jax.experimental import pallas as pl
from jax.experimental.pallas import tpu as pltpu
from jax.experimental.pallas import tpu_sc as plsc

SEQ = 8192
DM = 1024
NCLS = 42
NCLS_PAD = 48
NSUB = 16
KCH = 2 * DM // NSUB
MCH = SEQ // NSUB
SUBROW = 128
ROWS_PER = DM // SUBROW
NEG_BIG = -1e30


def _iota16():
    return lax.broadcasted_iota(jnp.int32, (16,), 0)


def _lane_allsum(v, scratch):
    for s in (1, 2, 4, 8):
        scratch[...] = v
        v = v + plsc.load_gather(scratch, [jnp.bitwise_xor(_iota16(), s)])
    return v


def _lane_allmax(v, scratch):
    for s in (1, 2, 4, 8):
        scratch[...] = v
        v = jnp.maximum(
            v, plsc.load_gather(scratch, [jnp.bitwise_xor(_iota16(), s)]))
    return v


ROWW = 80


def _body(bert_hbm, e1_hbm, e2_hbm, w_hbm, b_hbm, out_hbm,
          w_v, x_v, m_v, pv_v, all_v, part_v, b_v, out_v, red_v,
          sh, sem_w, sem_g):
    cid = lax.axis_index("c")
    sid = lax.axis_index("s")
    iota = _iota16()
    zeros16 = jnp.zeros((16,), jnp.int32)

    pltpu.sync_copy(e1_hbm.at[pl.ds(sid * MCH, MCH)], m_v.at[pl.ds(0, MCH)])
    pltpu.sync_copy(e2_hbm.at[pl.ds(sid * MCH, MCH)],
                    m_v.at[pl.ds(MCH, MCH)])
    wcopy = pltpu.async_copy(
        w_hbm.at[pl.ds(sid * (KCH * NCLS_PAD), KCH * NCLS_PAD)],
        w_v, sem_w)
    base = sid * MCH
    acc1 = zeros16
    acc2 = zeros16
    for j in range(MCH // 16):
        bv1 = m_v[pl.ds(16 * j, 16)] != 0
        bv2 = m_v[pl.ds(MCH + 16 * j, 16)] != 0
        pc1 = plsc.all_reduce_population_count(bv1)
        pc2 = plsc.all_reduce_population_count(bv2)
        f1 = plsc.all_reduce_ffs(bv1)
        f2 = plsc.all_reduce_ffs(bv2)
        acc1 = acc1 + jnp.where(pc1 > 0, f1 + (base + 16 * j), 0)
        acc2 = acc2 + jnp.where(pc2 > 0, f2 + (base + 16 * j), 0)
    pv_v[pl.ds(0, 16)] = acc1
    pv_v[pl.ds(16, 16)] = acc2
    pltpu.sync_copy(pv_v, sh.at[pl.ds(sid * ROWW, 32)])
    plsc.subcore_barrier()

    pltpu.sync_copy(sh, all_v)
    tot1 = zeros16
    tot2 = zeros16
    for j in range(NSUB):
        tot1 = tot1 + all_v[pl.ds(j * ROWW, 16)]
        tot2 = tot2 + all_v[pl.ds(j * ROWW + 16, 16)]

    half = NSUB // 2
    ridx = jnp.where(sid < half, tot1 * ROWS_PER + sid,
                     tot2 * ROWS_PER + (sid - half))
    pltpu.async_copy(bert_hbm.at[ridx], x_v, sem_g).wait()

    wcopy.wait()
    l0 = jnp.zeros((16,), jnp.float32)
    l1 = jnp.zeros((16,), jnp.float32)
    l2 = jnp.zeros((16,), jnp.float32)
    ones16 = jnp.full((16,), 1, jnp.int32)
    for k in range(KCH):
        xs = plsc.load_gather(
            x_v, [ones16, jnp.full((16,), k, jnp.int32)])
        woff = k * NCLS_PAD
        l0 = l0 + xs * w_v[pl.ds(woff, 16)]
        l1 = l1 + xs * w_v[pl.ds(woff + 16, 16)]
        l2 = l2 + xs * w_v[pl.ds(woff + 32, 16)]
    part_v[pl.ds(0, 16)] = plsc.bitcast(l0, jnp.int32)
    part_v[pl.ds(16, 16)] = plsc.bitcast(l1, jnp.int32)
    part_v[pl.ds(32, 16)] = plsc.bitcast(l2, jnp.int32)
    pltpu.sync_copy(part_v, sh.at[pl.ds(sid * ROWW + 32, 48)])
    plsc.subcore_barrier()

    @pl.when(jnp.logical_and(cid == 0, sid == 0))
    def _():
        pltpu.sync_copy(b_hbm, b_v)
        pltpu.sync_copy(sh, all_v)
        g0 = jnp.zeros((16,), jnp.float32)
        g1 = jnp.zeros((16,), jnp.float32)
        g2 = jnp.zeros((16,), jnp.float32)
        for j in range(NSUB):
            g0 = g0 + plsc.bitcast(all_v[pl.ds(j * ROWW + 32, 16)],
                                   jnp.float32)
            g1 = g1 + plsc.bitcast(all_v[pl.ds(j * ROWW + 48, 16)],
                                   jnp.float32)
            g2 = g2 + plsc.bitcast(all_v[pl.ds(j * ROWW + 64, 16)],
                                   jnp.float32)
        g0 = g0 + b_v[pl.ds(0, 16)]
        g1 = g1 + b_v[pl.ds(16, 16)]
        g2 = jnp.where(iota < NCLS - 32, g2 + b_v[pl.ds(32, 16)], NEG_BIG)
        m = _lane_allmax(jnp.maximum(jnp.maximum(g0, g1), g2), red_v)
        e0 = jnp.exp(g0 - m)
        e1 = jnp.exp(g1 - m)
        e2 = jnp.exp(g2 - m)
        s = _lane_allsum(e0 + e1 + e2, red_v)
        out_v[pl.ds(0, 16)] = e0 / s
        out_v[pl.ds(16, 16)] = e1 / s
        out_v[pl.ds(32, 16)] = e2 / s
        pltpu.sync_copy(out_v, out_hbm)


@jax.jit
def _entity_start_head_sc(bert2, e1i, e2i, w_pad, b_pad):
    mesh = plsc.VectorSubcoreMesh(core_axis_name="c", subcore_axis_name="s",
                                  num_cores=1, num_subcores=NSUB)
    return pl.kernel(
        _body,
        out_type=jax.ShapeDtypeStruct((NCLS_PAD,), jnp.float32),
        mesh=mesh,
        compiler_params=pltpu.CompilerParams(needs_layout_passes=False),
        scratch_types=[
            pltpu.VMEM((KCH * NCLS_PAD,), jnp.float32),
            pltpu.VMEM((16, SUBROW), jnp.float32),
            pltpu.VMEM((2 * MCH,), jnp.int32),
            pltpu.VMEM((32,), jnp.int32),
            pltpu.VMEM((NSUB * ROWW,), jnp.int32),
            pltpu.VMEM((NCLS_PAD,), jnp.int32),
            pltpu.VMEM((NCLS_PAD,), jnp.float32),
            pltpu.VMEM((NCLS_PAD,), jnp.float32),
            pltpu.VMEM((16,), jnp.float32),
            pltpu.VMEM_SHARED((NSUB * ROWW,), jnp.int32),
            pltpu.SemaphoreType.DMA,
            pltpu.SemaphoreType.DMA,
        ],
    )(bert2, e1i, e2i, w_pad, b_pad)


def kernel(bert_output, e1_mask, e2_mask, W, b):
    bert2 = bert_output.reshape(SEQ * ROWS_PER, SUBROW)
    e1i = e1_mask.reshape(SEQ).astype(jnp.int32)
    e2i = e2_mask.reshape(SEQ).astype(jnp.int32)
    w_pad = jnp.pad(W, ((0, 0), (0, NCLS_PAD - NCLS))).reshape(-1)
    b_pad = jnp.pad(b, (0, NCLS_PAD - NCLS))
    out = _entity_start_head_sc(bert2, e1i, e2i, w_pad, b_pad)
    return out[:NCLS].reshape(1, NCLS)


# speedup vs baseline: 1.0457x; 1.0193x over previous
import jax
import jax.numpy as jnp
from jax import lax
from jax.experimental import pallas as pl
from jax.experimental.pallas import tpu as pltpu
from jax.experimental.pallas import tpu_sc as plsc

SEQ = 8192
DM = 1024
NCLS = 42
NCLS_PAD = 48
NSUB = 16
KCH = 2 * DM // NSUB
MCH = SEQ // NSUB
SUBROW = 128
ROWS_PER = DM // SUBROW
NEG_BIG = -1e30


def _iota16():
    return lax.broadcasted_iota(jnp.int32, (16,), 0)


def _lane_allsum(v, scratch):
    for s in (1, 2, 4, 8):
        scratch[...] = v
        v = v + plsc.load_gather(scratch, [jnp.bitwise_xor(_iota16(), s)])
    return v


def _lane_allsum_i32(v, scratch):
    for s in (1, 2, 4, 8):
        scratch[...] = v
        v = v + plsc.load_gather(scratch, [jnp.bitwise_xor(_iota16(), s)])
    return v


def _lane_allmax(v, scratch):
    for s in (1, 2, 4, 8):
        scratch[...] = v
        v = jnp.maximum(
            v, plsc.load_gather(scratch, [jnp.bitwise_xor(_iota16(), s)]))
    return v


ROWW = 80


def _body(bert_hbm, e1_hbm, e2_hbm, w_hbm, b_hbm, out_hbm,
          w_v, x_v, m_v, pv_v, all_v, part_v, b_v, out_v, red_v, ired_v,
          sh, sem_w, sem_g):
    cid = lax.axis_index("c")
    sid = lax.axis_index("s")
    iota = _iota16()
    zeros16 = jnp.zeros((16,), jnp.int32)

    pltpu.sync_copy(e1_hbm.at[pl.ds(sid * MCH, MCH)], m_v.at[pl.ds(0, MCH)])
    pltpu.sync_copy(e2_hbm.at[pl.ds(sid * MCH, MCH)],
                    m_v.at[pl.ds(MCH, MCH)])
    wcopy = pltpu.async_copy(
        w_hbm.at[pl.ds(sid * (KCH * NCLS_PAD), KCH * NCLS_PAD)],
        w_v, sem_w)
    base = sid * MCH

    def _mask_step(j, accs):
        a1, a2 = accs
        lv = iota + (base + 16 * j)
        a1 = a1 + m_v[pl.ds(16 * j, 16)] * lv
        a2 = a2 + m_v[pl.ds(MCH + 16 * j, 16)] * lv
        return (a1, a2)

    acc1, acc2 = lax.fori_loop(0, MCH // 16, _mask_step,
                               (zeros16, zeros16))
    pv_v[pl.ds(0, 16)] = acc1
    pv_v[pl.ds(16, 16)] = acc2
    pltpu.sync_copy(pv_v, sh.at[pl.ds(sid * ROWW, 32)])
    plsc.subcore_barrier()

    pltpu.sync_copy(sh, all_v)

    def _tot_step(j, tots):
        t1, t2 = tots
        t1 = t1 + all_v[pl.ds(j * ROWW, 16)]
        t2 = t2 + all_v[pl.ds(j * ROWW + 16, 16)]
        return (t1, t2)

    tot1, tot2 = lax.fori_loop(0, NSUB, _tot_step, (zeros16, zeros16))
    tot1 = _lane_allsum_i32(tot1, ired_v)
    tot2 = _lane_allsum_i32(tot2, ired_v)

    half = NSUB // 2
    ridx = jnp.where(sid < half, tot1 * ROWS_PER + sid,
                     tot2 * ROWS_PER + (sid - half))
    pltpu.async_copy(bert_hbm.at[ridx], x_v, sem_g).wait()

    wcopy.wait()
    l0 = jnp.zeros((16,), jnp.float32)
    l1 = jnp.zeros((16,), jnp.float32)
    l2 = jnp.zeros((16,), jnp.float32)
    ones16 = jnp.full((16,), 1, jnp.int32)

    def _mm_step(j, accs):
        a0, a1, a2 = accs
        for l in range(16):
            kk = j * 16 + l
            xs = plsc.load_gather(x_v, [ones16, zeros16 + kk])
            woff = kk * NCLS_PAD
            a0 = a0 + xs * w_v[pl.ds(woff, 16)]
            a1 = a1 + xs * w_v[pl.ds(woff + 16, 16)]
            a2 = a2 + xs * w_v[pl.ds(woff + 32, 16)]
        return (a0, a1, a2)

    l0, l1, l2 = lax.fori_loop(0, KCH // 16, _mm_step, (l0, l1, l2))
    part_v[pl.ds(0, 16)] = plsc.bitcast(l0, jnp.int32)
    part_v[pl.ds(16, 16)] = plsc.bitcast(l1, jnp.int32)
    part_v[pl.ds(32, 16)] = plsc.bitcast(l2, jnp.int32)
    pltpu.sync_copy(part_v, sh.at[pl.ds(sid * ROWW + 32, 48)])
    plsc.subcore_barrier()

    @pl.when(jnp.logical_and(cid == 0, sid == 0))
    def _():
        pltpu.sync_copy(b_hbm, b_v)
        pltpu.sync_copy(sh, all_v)
        def _g_step(j, gs):
            g0, g1, g2 = gs
            g0 = g0 + plsc.bitcast(all_v[pl.ds(j * ROWW + 32, 16)],
                                   jnp.float32)
            g1 = g1 + plsc.bitcast(all_v[pl.ds(j * ROWW + 48, 16)],
                                   jnp.float32)
            g2 = g2 + plsc.bitcast(all_v[pl.ds(j * ROWW + 64, 16)],
                                   jnp.float32)
            return (g0, g1, g2)

        z16f = jnp.zeros((16,), jnp.float32)
        g0, g1, g2 = lax.fori_loop(0, NSUB, _g_step, (z16f, z16f, z16f))
        g0 = g0 + b_v[pl.ds(0, 16)]
        g1 = g1 + b_v[pl.ds(16, 16)]
        g2 = jnp.where(iota < NCLS - 32, g2 + b_v[pl.ds(32, 16)], NEG_BIG)
        m = _lane_allmax(jnp.maximum(jnp.maximum(g0, g1), g2), red_v)
        e0 = jnp.exp(g0 - m)
        e1 = jnp.exp(g1 - m)
        e2 = jnp.exp(g2 - m)
        s = _lane_allsum(e0 + e1 + e2, red_v)
        out_v[pl.ds(0, 16)] = e0 / s
        out_v[pl.ds(16, 16)] = e1 / s
        out_v[pl.ds(32, 16)] = e2 / s
        pltpu.sync_copy(out_v, out_hbm)


@jax.jit
def _entity_start_head_sc(bert2, e1i, e2i, w_pad, b_pad):
    mesh = plsc.VectorSubcoreMesh(core_axis_name="c", subcore_axis_name="s",
                                  num_cores=1, num_subcores=NSUB)
    return pl.kernel(
        _body,
        out_type=jax.ShapeDtypeStruct((NCLS_PAD,), jnp.float32),
        mesh=mesh,
        compiler_params=pltpu.CompilerParams(needs_layout_passes=False),
        scratch_types=[
            pltpu.VMEM((KCH * NCLS_PAD,), jnp.float32),
            pltpu.VMEM((16, SUBROW), jnp.float32),
            pltpu.VMEM((2 * MCH,), jnp.int32),
            pltpu.VMEM((32,), jnp.int32),
            pltpu.VMEM((NSUB * ROWW,), jnp.int32),
            pltpu.VMEM((NCLS_PAD,), jnp.int32),
            pltpu.VMEM((NCLS_PAD,), jnp.float32),
            pltpu.VMEM((NCLS_PAD,), jnp.float32),
            pltpu.VMEM((16,), jnp.float32),
            pltpu.VMEM((16,), jnp.int32),
            pltpu.VMEM_SHARED((NSUB * ROWW,), jnp.int32),
            pltpu.SemaphoreType.DMA,
            pltpu.SemaphoreType.DMA,
        ],
    )(bert2, e1i, e2i, w_pad, b_pad)


def kernel(bert_output, e1_mask, e2_mask, W, b):
    bert2 = bert_output.reshape(SEQ * ROWS_PER, SUBROW)
    e1i = e1_mask.reshape(SEQ).astype(jnp.int32)
    e2i = e2_mask.reshape(SEQ).astype(jnp.int32)
    w_pad = jnp.pad(W, ((0, 0), (0, NCLS_PAD - NCLS))).reshape(-1)
    b_pad = jnp.pad(b, (0, NCLS_PAD - NCLS))
    out = _entity_start_head_sc(bert2, e1i, e2i, w_pad, b_pad)
    return out[:NCLS].reshape(1, NCLS)
